# baseline (speedup 1.0000x reference)
import functools

import jax
import jax.numpy as jnp
from jax import lax
from jax.experimental import pallas as pl
from jax.experimental.pallas import tpu as pltpu, tpu_sc as plsc

_N = 10000
_NP = 10240
_E = 160000
_NC = 2
_NS = 16
_NW = _NC * _NS
_C = 128
_EPT = _E // _NS
_NCH = 80
_EPW = _NCH * _C
_RPT = _NP // _NS

_SC_PARAMS = pltpu.CompilerParams(use_tc_tiling_on_sc=False,
                                  needs_layout_passes=False)
_MESH = plsc.VectorSubcoreMesh(core_axis_name="c", subcore_axis_name="s")


def _make_prop(D, nhops):
    out_type = jax.ShapeDtypeStruct((nhops, _NC, _NP, D), jnp.float32)
    scratch = [
        pltpu.VMEM((_NCH, _C), jnp.int32),
        pltpu.VMEM((_NCH, _C), jnp.int32),
        pltpu.VMEM((_EPW,), jnp.float32),
        pltpu.VMEM((_C, D), jnp.float32),
        pltpu.VMEM((_C, D), jnp.float32),
        pltpu.VMEM_SHARED((_NP, D), jnp.float32),
        pltpu.SemaphoreType.DMA,
        pltpu.SemaphoreType.DMA,
    ]

    @functools.partial(pl.kernel, out_type=out_type, mesh=_MESH,
                       scratch_types=scratch, compiler_params=_SC_PARAMS)
    def prop(x, src, dst, w, zrows, out, src_v, dst_v, w_v, rows_a, rows_b,
             acc, sem_a, sem_b):
        cid = lax.axis_index("c")
        sid = lax.axis_index("s")
        wid = cid * _NS + sid
        rows = pl.ds(sid * _RPT, _RPT)
        pltpu.sync_copy(zrows, acc.at[rows])
        pltpu.sync_copy(src.at[wid], src_v)
        pltpu.sync_copy(dst.at[wid], dst_v)
        pltpu.sync_copy(w.at[wid, 0], w_v)
        plsc.subcore_barrier()

        def hop(table):
            def process(ci, buf, sem):
                pltpu.make_async_copy(table.at[src_v.at[ci]], buf, sem).wait()
                for g in range(_C // 16):
                    w16 = w_v[pl.ds(ci * _C + g * 16, 16)]
                    for ee in range(16):
                        e = g * 16 + ee
                        wspl = lax.gather(
                            w16, jnp.full((16, 1), ee, jnp.int32),
                            lax.GatherDimensionNumbers(
                                offset_dims=(), collapsed_slice_dims=(0,),
                                start_index_map=(0,)),
                            (1,), mode=lax.GatherScatterMode.PROMISE_IN_BOUNDS)
                        for j in range(D // 16):
                            sl = pl.ds(j * 16, 16)
                            buf[e, sl] = buf[e, sl] * wspl
                pltpu.sync_copy(buf, acc.at[dst_v.at[ci]], add=True)

            def pair(cp, carry):
                ci = cp * 2
                pltpu.async_copy(table.at[src_v.at[ci + 1]], rows_b, sem_b)
                process(ci, rows_a, sem_a)

                @pl.when(ci + 2 < _NCH)
                def _():
                    pltpu.async_copy(table.at[src_v.at[ci + 2]], rows_a, sem_a)
                process(ci + 1, rows_b, sem_b)
                return carry

            pltpu.async_copy(table.at[src_v.at[0]], rows_a, sem_a)
            lax.fori_loop(0, _NCH // 2, pair, 0)

        hop(x)
        plsc.subcore_barrier()
        pltpu.sync_copy(acc.at[rows], out.at[0, cid, rows])
        if nhops == 2:
            plsc.subcore_barrier()
            pltpu.sync_copy(zrows, acc.at[rows])
            plsc.subcore_barrier()
            hop(out.at[0, cid])
            plsc.subcore_barrier()
            pltpu.sync_copy(acc.at[rows], out.at[1, cid, rows])

    return prop


_prop_dec = _make_prop(64, 1)
_prop_cell = _make_prop(80, 2)


def _make_prep():
    out_type = (jax.ShapeDtypeStruct((_NW, 1, _EPW), jnp.float32),
                jax.ShapeDtypeStruct((_NW, 1, _EPW), jnp.float32))
    scratch = [
        pltpu.VMEM((_NCH, _C), jnp.int32),
        pltpu.VMEM((_NCH, _C), jnp.int32),
        pltpu.VMEM((_EPW,), jnp.float32),
        pltpu.VMEM((_EPW,), jnp.float32),
        pltpu.VMEM((_RPT,), jnp.float32),
        pltpu.VMEM((_C,), jnp.float32),
        pltpu.VMEM((_C,), jnp.float32),
        pltpu.VMEM_SHARED((_NP,), jnp.float32),
        pltpu.VMEM_SHARED((_NP,), jnp.float32),
        pltpu.SemaphoreType.DMA,
    ]

    @functools.partial(pl.kernel, out_type=out_type, mesh=_MESH,
                       scratch_types=scratch, compiler_params=_SC_PARAMS)
    def prep(nidx, oidx, ew, zdeg, w_cell, w_dec, n_v, o_v, ew_v, ewd_v,
             t_v, gc_v, gd_v, deg_c, deg_d, sem):
        cid = lax.axis_index("c")
        sid = lax.axis_index("s")
        wid = cid * _NS + sid
        myslice = pl.ds(sid * _RPT, _RPT)
        pltpu.sync_copy(zdeg, deg_c.at[myslice])
        pltpu.sync_copy(zdeg, deg_d.at[myslice])
        pltpu.sync_copy(nidx.at[wid], n_v)
        pltpu.sync_copy(oidx.at[wid], o_v)
        pltpu.sync_copy(ew.at[wid, 0], ew_v)

        def mk_ewd(ci, carry):
            for g in range(_C // 16):
                sl = pl.ds(ci * _C + g * 16, 16)
                n16 = n_v[ci, pl.ds(g * 16, 16)]
                o16 = o_v[ci, pl.ds(g * 16, 16)]
                ewd_v[sl] = jnp.where(o16 == n16, 0.0, ew_v[sl])
            return carry
        lax.fori_loop(0, _NCH, mk_ewd, 0)
        plsc.subcore_barrier()

        def deg_add(ci, carry):
            pltpu.sync_copy(ew_v.at[pl.ds(ci * _C, _C)],
                            deg_c.at[n_v.at[ci]], add=True)
            pltpu.sync_copy(ewd_v.at[pl.ds(ci * _C, _C)],
                            deg_d.at[n_v.at[ci]], add=True)
            return carry
        lax.fori_loop(0, _NCH, deg_add, 0)
        plsc.subcore_barrier()

        for deg in (deg_c, deg_d):
            pltpu.sync_copy(deg.at[myslice], t_v)
            for g in range(_RPT // 16):
                sl = pl.ds(g * 16, 16)
                d16 = t_v[sl]
                t_v[sl] = jnp.where(d16 > 0.0, 1.0 / d16, 0.0)
            pltpu.sync_copy(t_v, deg.at[myslice])
        plsc.subcore_barrier()

        def wmul(ci, carry):
            pltpu.async_copy(deg_c.at[n_v.at[ci]], gc_v, sem).wait()
            pltpu.async_copy(deg_d.at[n_v.at[ci]], gd_v, sem).wait()
            for g in range(_C // 16):
                sl = pl.ds(ci * _C + g * 16, 16)
                sg = pl.ds(g * 16, 16)
                ew_v[sl] = ew_v[sl] * gc_v[sg]
                ewd_v[sl] = ewd_v[sl] * gd_v[sg]
            return carry
        lax.fori_loop(0, _NCH, wmul, 0)
        pltpu.sync_copy(ew_v, w_cell.at[wid, 0])
        pltpu.sync_copy(ewd_v, w_dec.at[wid, 0])

    return prep


_prep = _make_prep()


def _pad_blocks(a, fill):
    a = a.reshape(_NS, _EPT)
    pad = jnp.full((_NS, _EPW - _EPT), fill, a.dtype)
    return jnp.concatenate([a, pad], 1)


def kernel(x, mask, edge_weight, edge_index, h0, W_fs, b_fs, W_r, b_r, W_u, b_u,
           W_c, b_c, W_lin_in, b_lin_in, W_gc, b_gc, W_lin_out, b_lin_out,
           W_ro, b_ro, prelu_a):
    B, T, N, F = x.shape
    H = h0.shape[-1]
    src0, dst0 = edge_index[0], edge_index[1]

    def stack2(f, b):
        return jnp.stack([_pad_blocks(f, 0), _pad_blocks(b, 0)]) \
                  .reshape(_NW, _NCH, _C)

    SRC = stack2(src0, dst0)
    DST = stack2(dst0, src0)
    EW = jnp.stack([_pad_blocks(edge_weight, 0.0)] * 2).reshape(_NW, 1, _EPW)
    zdeg = jnp.zeros((_RPT,), jnp.float32)
    W_CELL, W_DEC = _prep(DST, SRC, EW, zdeg)
    zr64 = jnp.zeros((_RPT, 64), jnp.float32)
    zr80 = jnp.zeros((_RPT, 80), jnp.float32)

    D_IN = 2 * F + H

    def cell_props(v):
        vp = jnp.pad(v, ((0, _NP - _N), (0, 80 - D_IN)))
        o = _prop_cell(vp, SRC, DST, W_CELL, zr80)
        return o[0, 0, :_N, :D_IN], o[1, 0, :_N, :D_IN], \
               o[0, 1, :_N, :D_IN], o[1, 1, :_N, :D_IN]

    h = jnp.broadcast_to(h0, (N, H))
    imps, preds, states, reprs = [], [], [], []
    for t in range(T):
        x_s = x[0, t]
        m_s = mask[0, t]
        h_s = h
        xh1 = h_s @ W_fs + b_fs
        x_s = jnp.where(m_s != 0, x_s, xh1)
        z = jnp.concatenate([x_s, m_s, h_s], -1) @ W_lin_in + b_lin_in
        zo = _prop_dec(jnp.pad(z, ((0, _NP - _N), (0, 0))),
                       SRC, DST, W_DEC, zr64)
        dec = jnp.concatenate([zo[0, 0, :_N], zo[0, 1, :_N]], -1) @ W_gc + b_gc
        dec = jnp.concatenate([dec, h_s], -1) @ W_lin_out + b_lin_out
        dec = jnp.where(dec >= 0, dec, prelu_a * dec)
        rep = jnp.concatenate([dec, h_s], -1)
        xh2 = rep @ W_ro + b_ro
        x_s = jnp.where(m_s != 0, x_s, xh2)
        inp = jnp.concatenate([x_s, m_s], -1)
        xh = jnp.concatenate([inp, h], -1)
        f1, f2, b1, b2 = cell_props(xh)
        ru_in = jnp.concatenate([xh, f1, f2, b1, b2], -1)
        r = jax.nn.sigmoid(ru_in @ W_r + b_r)
        u = jax.nn.sigmoid(ru_in @ W_u + b_u)
        xc = jnp.concatenate([inp, r * h], -1)
        g1, g2, g3, g4 = cell_props(xc)
        c = jnp.tanh(jnp.concatenate([xc, g1, g2, g3, g4], -1) @ W_c + b_c)
        h = u * h + (1.0 - u) * c
        imps.append(xh2)
        preds.append(xh1)
        states.append(h)
        reprs.append(rep)
    st = lambda xs: jnp.stack(xs, 0)[None]
    return (st(imps), st(preds), st(reprs), st(states)[:, :, None])


# speedup vs baseline: 1.1147x; 1.1147x over previous
import functools

import jax
import jax.numpy as jnp
from jax import lax
from jax.experimental import pallas as pl
from jax.experimental.pallas import tpu as pltpu, tpu_sc as plsc

_N = 10000
_NP = 10240
_E = 160000
_NC = 2
_NS = 16
_NW = _NC * _NS
_C = 128
_EPT = _E // _NS
_NCH = 80
_EPW = _NCH * _C
_RPT = _NP // _NS

_SC_PARAMS = pltpu.CompilerParams(use_tc_tiling_on_sc=False,
                                  needs_layout_passes=False)
_MESH = plsc.VectorSubcoreMesh(core_axis_name="c", subcore_axis_name="s")


def _make_prop(D, nhops, spmem_table):
    out_type = jax.ShapeDtypeStruct((nhops, _NC, _NP, D), jnp.float32)
    scratch = [
        pltpu.VMEM((_NCH, _C), jnp.int32),
        pltpu.VMEM((_NCH, _C), jnp.int32),
        pltpu.VMEM((_EPW,), jnp.float32),
        pltpu.VMEM((_C, D), jnp.float32),
        pltpu.VMEM((_C, D), jnp.float32),
        pltpu.VMEM_SHARED((_NP, D), jnp.float32),
        pltpu.SemaphoreType.DMA,
        pltpu.SemaphoreType.DMA,
    ]
    if spmem_table:
        scratch.insert(6, pltpu.VMEM_SHARED((_NP, D), jnp.float32))

    @functools.partial(pl.kernel, out_type=out_type, mesh=_MESH,
                       scratch_types=scratch, compiler_params=_SC_PARAMS)
    def prop(x, src, dst, w, zrows, out, src_v, dst_v, w_v, rows_a, rows_b,
             *rest):
        if spmem_table:
            spm_a, spm_b, sem_a, sem_b = rest
        else:
            spm_a, sem_a, sem_b = rest
        cid = lax.axis_index("c")
        sid = lax.axis_index("s")
        wid = cid * _NS + sid
        rows = pl.ds(sid * _RPT, _RPT)
        if spmem_table:
            pltpu.sync_copy(x.at[rows], spm_b.at[rows])
        pltpu.sync_copy(zrows, spm_a.at[rows])
        pltpu.sync_copy(src.at[wid], src_v)
        pltpu.sync_copy(dst.at[wid], dst_v)
        pltpu.sync_copy(w.at[wid, 0], w_v)
        plsc.subcore_barrier()

        def hop(table, acc):
            def process(ci, buf, sem):
                pltpu.make_async_copy(table.at[src_v.at[ci]], buf, sem).wait()
                for g in range(_C // 16):
                    w16 = w_v[pl.ds(ci * _C + g * 16, 16)]
                    for ee in range(16):
                        e = g * 16 + ee
                        wspl = lax.gather(
                            w16, jnp.full((16, 1), ee, jnp.int32),
                            lax.GatherDimensionNumbers(
                                offset_dims=(), collapsed_slice_dims=(0,),
                                start_index_map=(0,)),
                            (1,), mode=lax.GatherScatterMode.PROMISE_IN_BOUNDS)
                        for j in range(D // 16):
                            sl = pl.ds(j * 16, 16)
                            buf[e, sl] = buf[e, sl] * wspl
                pltpu.sync_copy(buf, acc.at[dst_v.at[ci]], add=True)

            def pair(cp, carry):
                ci = cp * 2
                pltpu.async_copy(table.at[src_v.at[ci + 1]], rows_b, sem_b)
                process(ci, rows_a, sem_a)

                @pl.when(ci + 2 < _NCH)
                def _():
                    pltpu.async_copy(table.at[src_v.at[ci + 2]], rows_a, sem_a)
                process(ci + 1, rows_b, sem_b)
                return carry

            pltpu.async_copy(table.at[src_v.at[0]], rows_a, sem_a)
            lax.fori_loop(0, _NCH // 2, pair, 0)

        if spmem_table:
            hop(spm_b, spm_a)
            plsc.subcore_barrier()
            pltpu.sync_copy(spm_a.at[rows], out.at[0, cid, rows])
            if nhops == 2:
                pltpu.sync_copy(zrows, spm_b.at[rows])
                plsc.subcore_barrier()
                hop(spm_a, spm_b)
                plsc.subcore_barrier()
                pltpu.sync_copy(spm_b.at[rows], out.at[1, cid, rows])
        else:
            hop(x, spm_a)
            plsc.subcore_barrier()
            pltpu.sync_copy(spm_a.at[rows], out.at[0, cid, rows])
            if nhops == 2:
                plsc.subcore_barrier()
                pltpu.sync_copy(zrows, spm_a.at[rows])
                plsc.subcore_barrier()
                hop(out.at[0, cid], spm_a)
                plsc.subcore_barrier()
                pltpu.sync_copy(spm_a.at[rows], out.at[1, cid, rows])

    return prop


_prop_dec = _make_prop(64, 1, spmem_table=True)
_prop_cell = _make_prop(80, 2, spmem_table=False)


def _make_prep():
    out_type = (jax.ShapeDtypeStruct((_NW, 1, _EPW), jnp.float32),
                jax.ShapeDtypeStruct((_NW, 1, _EPW), jnp.float32))
    scratch = [
        pltpu.VMEM((_NCH, _C), jnp.int32),
        pltpu.VMEM((_NCH, _C), jnp.int32),
        pltpu.VMEM((_EPW,), jnp.float32),
        pltpu.VMEM((_EPW,), jnp.float32),
        pltpu.VMEM((_RPT,), jnp.float32),
        pltpu.VMEM((_C,), jnp.float32),
        pltpu.VMEM((_C,), jnp.float32),
        pltpu.VMEM_SHARED((_NP,), jnp.float32),
        pltpu.VMEM_SHARED((_NP,), jnp.float32),
        pltpu.SemaphoreType.DMA,
    ]

    @functools.partial(pl.kernel, out_type=out_type, mesh=_MESH,
                       scratch_types=scratch, compiler_params=_SC_PARAMS)
    def prep(nidx, oidx, ew, zdeg, w_cell, w_dec, n_v, o_v, ew_v, ewd_v,
             t_v, gc_v, gd_v, deg_c, deg_d, sem):
        cid = lax.axis_index("c")
        sid = lax.axis_index("s")
        wid = cid * _NS + sid
        myslice = pl.ds(sid * _RPT, _RPT)
        pltpu.sync_copy(zdeg, deg_c.at[myslice])
        pltpu.sync_copy(zdeg, deg_d.at[myslice])
        pltpu.sync_copy(nidx.at[wid], n_v)
        pltpu.sync_copy(oidx.at[wid], o_v)
        pltpu.sync_copy(ew.at[wid, 0], ew_v)

        def mk_ewd(ci, carry):
            for g in range(_C // 16):
                sl = pl.ds(ci * _C + g * 16, 16)
                n16 = n_v[ci, pl.ds(g * 16, 16)]
                o16 = o_v[ci, pl.ds(g * 16, 16)]
                ewd_v[sl] = jnp.where(o16 == n16, 0.0, ew_v[sl])
            return carry
        lax.fori_loop(0, _NCH, mk_ewd, 0)
        plsc.subcore_barrier()

        def deg_add(ci, carry):
            pltpu.sync_copy(ew_v.at[pl.ds(ci * _C, _C)],
                            deg_c.at[n_v.at[ci]], add=True)
            pltpu.sync_copy(ewd_v.at[pl.ds(ci * _C, _C)],
                            deg_d.at[n_v.at[ci]], add=True)
            return carry
        lax.fori_loop(0, _NCH, deg_add, 0)
        plsc.subcore_barrier()

        for deg in (deg_c, deg_d):
            pltpu.sync_copy(deg.at[myslice], t_v)
            for g in range(_RPT // 16):
                sl = pl.ds(g * 16, 16)
                d16 = t_v[sl]
                t_v[sl] = jnp.where(d16 > 0.0, 1.0 / d16, 0.0)
            pltpu.sync_copy(t_v, deg.at[myslice])
        plsc.subcore_barrier()

        def wmul(ci, carry):
            pltpu.async_copy(deg_c.at[n_v.at[ci]], gc_v, sem).wait()
            pltpu.async_copy(deg_d.at[n_v.at[ci]], gd_v, sem).wait()
            for g in range(_C // 16):
                sl = pl.ds(ci * _C + g * 16, 16)
                sg = pl.ds(g * 16, 16)
                ew_v[sl] = ew_v[sl] * gc_v[sg]
                ewd_v[sl] = ewd_v[sl] * gd_v[sg]
            return carry
        lax.fori_loop(0, _NCH, wmul, 0)
        pltpu.sync_copy(ew_v, w_cell.at[wid, 0])
        pltpu.sync_copy(ewd_v, w_dec.at[wid, 0])

    return prep


_prep = _make_prep()


def _pad_blocks(a, fill):
    a = a.reshape(_NS, _EPT)
    pad = jnp.full((_NS, _EPW - _EPT), fill, a.dtype)
    return jnp.concatenate([a, pad], 1)


def kernel(x, mask, edge_weight, edge_index, h0, W_fs, b_fs, W_r, b_r, W_u, b_u,
           W_c, b_c, W_lin_in, b_lin_in, W_gc, b_gc, W_lin_out, b_lin_out,
           W_ro, b_ro, prelu_a):
    B, T, N, F = x.shape
    H = h0.shape[-1]
    src0, dst0 = edge_index[0], edge_index[1]

    def stack2(f, b):
        return jnp.stack([_pad_blocks(f, 0), _pad_blocks(b, 0)]) \
                  .reshape(_NW, _NCH, _C)

    SRC = stack2(src0, dst0)
    DST = stack2(dst0, src0)
    EW = jnp.stack([_pad_blocks(edge_weight, 0.0)] * 2).reshape(_NW, 1, _EPW)
    zdeg = jnp.zeros((_RPT,), jnp.float32)
    W_CELL, W_DEC = _prep(DST, SRC, EW, zdeg)
    zr64 = jnp.zeros((_RPT, 64), jnp.float32)
    zr80 = jnp.zeros((_RPT, 80), jnp.float32)

    D_IN = 2 * F + H

    def cell_props(v):
        vp = jnp.pad(v, ((0, _NP - _N), (0, 80 - D_IN)))
        o = _prop_cell(vp, SRC, DST, W_CELL, zr80)
        return o[0, 0, :_N, :D_IN], o[1, 0, :_N, :D_IN], \
               o[0, 1, :_N, :D_IN], o[1, 1, :_N, :D_IN]

    h = jnp.broadcast_to(h0, (N, H))
    imps, preds, states, reprs = [], [], [], []
    for t in range(T):
        x_s = x[0, t]
        m_s = mask[0, t]
        h_s = h
        xh1 = h_s @ W_fs + b_fs
        x_s = jnp.where(m_s != 0, x_s, xh1)
        z = jnp.concatenate([x_s, m_s, h_s], -1) @ W_lin_in + b_lin_in
        zo = _prop_dec(jnp.pad(z, ((0, _NP - _N), (0, 0))),
                       SRC, DST, W_DEC, zr64)
        dec = jnp.concatenate([zo[0, 0, :_N], zo[0, 1, :_N]], -1) @ W_gc + b_gc
        dec = jnp.concatenate([dec, h_s], -1) @ W_lin_out + b_lin_out
        dec = jnp.where(dec >= 0, dec, prelu_a * dec)
        rep = jnp.concatenate([dec, h_s], -1)
        xh2 = rep @ W_ro + b_ro
        x_s = jnp.where(m_s != 0, x_s, xh2)
        inp = jnp.concatenate([x_s, m_s], -1)
        xh = jnp.concatenate([inp, h], -1)
        f1, f2, b1, b2 = cell_props(xh)
        ru_in = jnp.concatenate([xh, f1, f2, b1, b2], -1)
        r = jax.nn.sigmoid(ru_in @ W_r + b_r)
        u = jax.nn.sigmoid(ru_in @ W_u + b_u)
        xc = jnp.concatenate([inp, r * h], -1)
        g1, g2, g3, g4 = cell_props(xc)
        c = jnp.tanh(jnp.concatenate([xc, g1, g2, g3, g4], -1) @ W_c + b_c)
        h = u * h + (1.0 - u) * c
        imps.append(xh2)
        preds.append(xh1)
        states.append(h)
        reprs.append(rep)
    st = lambda xs: jnp.stack(xs, 0)[None]
    return (st(imps), st(preds), st(reprs), st(states)[:, :, None])
